# baseline (speedup 1.0000x reference)
import functools

import jax
import jax.numpy as jnp
from jax import lax
from jax.experimental import pallas as pl
from jax.experimental.pallas import tpu as pltpu
from jax.experimental.pallas import tpu_sc as plsc

_NC = 2
_NS = 16
_NW = _NC * _NS
_CH = 128
_H = 128

_mesh = plsc.VectorSubcoreMesh(core_axis_name="c", subcore_axis_name="s")


def _zero_rows(buf, nrows, ncols16):
    z = jnp.zeros((16,), jnp.float32)

    def body(i, _):
        r = i // ncols16
        k = i % ncols16
        buf[r, pl.ds(k * 16, 16)] = z
        return 0

    lax.fori_loop(0, nrows * ncols16, body, 0)


_NB = 16


def _make_spmm(n, cpw0, cpw1):
    npad = ((n + 8 * _NS - 1) // (8 * _NS)) * (8 * _NS)
    rpt = npad // _NS

    @functools.partial(
        pl.kernel,
        mesh=_mesh,
        out_type=jax.ShapeDtypeStruct((_NC, npad, _H), jnp.float32),
        scratch_types=[
            pltpu.VMEM((_NB, _CH), jnp.int32),
            pltpu.VMEM((_NB, _CH), jnp.int32),
            pltpu.VMEM((_CH, _H), jnp.float32),
            pltpu.VMEM((_CH, _H), jnp.float32),
            pltpu.VMEM_SHARED((npad, _H), jnp.float32),
            pltpu.SemaphoreType.DMA,
            pltpu.SemaphoreType.DMA,
        ],
    )
    def spmm_kernel(g_hbm, srca_hbm, dsta_hbm, srcb_hbm, dstb_hbm, out_hbm,
                    srcb, dstb, rows_a, rows_b, acc_sh, sem_a, sem_b):
        c = lax.axis_index("c")
        s = lax.axis_index("s")

        _zero_rows(rows_a, _CH, _H // 16)

        nfull = rpt // _CH
        rem = rpt - nfull * _CH
        for k in range(nfull):
            pltpu.sync_copy(rows_a, acc_sh.at[pl.ds(s * rpt + k * _CH, _CH)])
        if rem:
            pltpu.sync_copy(rows_a.at[pl.ds(0, rem)],
                            acc_sh.at[pl.ds(s * rpt + nfull * _CH, rem)])
        plsc.subcore_barrier()

        def gather_a(j):
            pltpu.async_copy(g_hbm.at[srcb.at[j]], rows_a, sem_a)

        def gather_b(j):
            pltpu.async_copy(g_hbm.at[srcb.at[j]], rows_b, sem_b)

        def wait_a(j):
            pltpu.make_async_copy(g_hbm.at[srcb.at[j]], rows_a, sem_a).wait()

        def wait_b(j):
            pltpu.make_async_copy(g_hbm.at[srcb.at[j]], rows_b, sem_b).wait()

        def scat_a(j):
            pltpu.sync_copy(rows_a, acc_sh.at[dstb.at[j]], add=True)

        def scat_b(j):
            pltpu.sync_copy(rows_b, acc_sh.at[dstb.at[j]], add=True)

        def run_edges(src_hbm, dst_hbm, nb):
            def batch_body(b, _):
                pltpu.sync_copy(src_hbm.at[s, pl.ds(b * _NB, _NB)], srcb)
                pltpu.sync_copy(dst_hbm.at[s, pl.ds(b * _NB, _NB)], dstb)

                gather_a(0)

                def pair_body(p, _):
                    gather_b(2 * p + 1)
                    wait_a(2 * p)
                    scat_a(2 * p)
                    gather_a(2 * p + 2)
                    wait_b(2 * p + 1)
                    scat_b(2 * p + 1)
                    return 0

                lax.fori_loop(0, _NB // 2 - 1, pair_body, 0)
                gather_b(_NB - 1)
                wait_a(_NB - 2)
                scat_a(_NB - 2)
                wait_b(_NB - 1)
                scat_b(_NB - 1)
                return 0

            lax.fori_loop(0, nb, batch_body, 0)

        @pl.when(c == 0)
        def _():
            run_edges(srca_hbm, dsta_hbm, cpw0 // _NB)

        @pl.when(c == 1)
        def _():
            run_edges(srcb_hbm, dstb_hbm, cpw1 // _NB)

        plsc.subcore_barrier()

        done = 0
        while done < rpt:
            oc = min(_CH, rpt - done)
            off = s * rpt + done
            pltpu.sync_copy(acc_sh.at[pl.ds(off, oc)], rows_a.at[pl.ds(0, oc)])
            pltpu.sync_copy(rows_a.at[pl.ds(0, oc)], out_hbm.at[c, pl.ds(off, oc)])
            done += oc

    return spmm_kernel


def _make_deg(n, cpw0, cpw1):
    npad = ((n + 8 * _NS - 1) // (8 * _NS)) * (8 * _NS)
    rpt = npad // _NS

    @functools.partial(
        pl.kernel,
        mesh=_mesh,
        out_type=jax.ShapeDtypeStruct((_NC, npad, _H), jnp.float32),
        scratch_types=[
            pltpu.VMEM((_NB, _CH), jnp.int32),
            pltpu.VMEM((_CH, _H), jnp.float32),
            pltpu.VMEM_SHARED((npad, _H), jnp.float32),
        ],
    )
    def deg_kernel(dsta_hbm, dstb_hbm, out_hbm, dstb, ones_v, acc_sh):
        c = lax.axis_index("c")
        s = lax.axis_index("s")

        _zero_rows(ones_v, _CH, _H // 16)

        nfull = rpt // _CH
        rem = rpt - nfull * _CH
        for k in range(nfull):
            pltpu.sync_copy(ones_v, acc_sh.at[pl.ds(s * rpt + k * _CH, _CH)])
        if rem:
            pltpu.sync_copy(ones_v.at[pl.ds(0, rem)],
                            acc_sh.at[pl.ds(s * rpt + nfull * _CH, rem)])

        one = jnp.ones((16,), jnp.float32)

        def fill_ones(i, _):
            r = i // (_H // 16)
            k = i % (_H // 16)
            ones_v[r, pl.ds(k * 16, 16)] = one
            return 0

        lax.fori_loop(0, _CH * (_H // 16), fill_ones, 0)
        plsc.subcore_barrier()

        def run_edges(dst_hbm, nb):
            def batch_body(b, _):
                pltpu.sync_copy(dst_hbm.at[s, pl.ds(b * _NB, _NB)], dstb)

                def body(j, _):
                    pltpu.sync_copy(ones_v, acc_sh.at[dstb.at[j]], add=True)
                    return 0

                lax.fori_loop(0, _NB, body, 0)
                return 0

            lax.fori_loop(0, nb, batch_body, 0)

        @pl.when(c == 0)
        def _():
            run_edges(dsta_hbm, cpw0 // _NB)

        @pl.when(c == 1)
        def _():
            run_edges(dstb_hbm, cpw1 // _NB)

        plsc.subcore_barrier()

        done = 0
        while done < rpt:
            oc = min(_CH, rpt - done)
            off = s * rpt + done
            pltpu.sync_copy(acc_sh.at[pl.ds(off, oc)], ones_v.at[pl.ds(0, oc)])
            pltpu.sync_copy(ones_v.at[pl.ds(0, oc)], out_hbm.at[c, pl.ds(off, oc)])
            done += oc

    return deg_kernel


_BLK = 400


def _mm1_body(x_ref, w_ref, o_ref):
    o_ref[...] = lax.dot_general(
        x_ref[...], w_ref[...], (((1,), (0,)), ((), ())),
        preferred_element_type=jnp.float32, precision=lax.Precision.HIGHEST)


def _scale_body(degp_ref, h_ref, dinv_ref, g_ref):
    deg = degp_ref[0, :, 0:1] + degp_ref[1, :, 0:1] + 1.0
    dinv = lax.rsqrt(deg)
    dinvb = jnp.broadcast_to(dinv, h_ref.shape)
    dinv_ref[...] = dinvb
    g_ref[...] = h_ref[...] * dinvb


def _mid_body(p_ref, g_ref, dinv_ref, b_ref, w_ref, o_ref):
    t = dinv_ref[...] * (p_ref[0] + p_ref[1] + g_ref[...]) + b_ref[...]
    t = jnp.maximum(t, 0.0)
    o_ref[...] = dinv_ref[...] * lax.dot_general(
        t, w_ref[...], (((1,), (0,)), ((), ())),
        preferred_element_type=jnp.float32, precision=lax.Precision.HIGHEST)


def _fin_body(p_ref, g_ref, dinv_ref, b_ref, w_ref, bf_ref, o_ref):
    t = dinv_ref[...] * (p_ref[0] + p_ref[1] + g_ref[...]) + b_ref[...]
    t = jnp.maximum(t, 0.0)
    o_ref[...] = lax.dot_general(
        t, w_ref[...], (((1,), (0,)), ((), ())),
        preferred_element_type=jnp.float32, precision=lax.Precision.HIGHEST) + bf_ref[...]


def _row_block(n, d):
    return pl.BlockSpec((_BLK, d), lambda i: (i, 0))


def _full_block(shape):
    nd = len(shape)
    return pl.BlockSpec(shape, lambda i, _nd=nd: (0,) * nd)


def kernel(x, edge_index, W1, b1, W2, b2, Wfc, bfc):
    n, d_in = x.shape
    e = edge_index.shape[1]
    grid = (n // _BLK,)

    chunks = -(-e // _CH)
    cpw = -(-chunks // _NW)
    cpw = ((cpw + _NB - 1) // _NB) * _NB
    cpw0 = ((2 * cpw * 85 // 100 + _NB - 1) // _NB) * _NB
    cpw1 = 2 * cpw - cpw0
    ep = _NW * cpw * _CH
    src = jnp.concatenate([edge_index[0], jnp.zeros((ep - e,), jnp.int32)])
    dst = jnp.concatenate([edge_index[1], jnp.full((ep - e,), n, jnp.int32)])
    na = _NS * cpw0 * _CH
    src3a = src[:na].reshape(_NS, cpw0, _CH)
    dst3a = dst[:na].reshape(_NS, cpw0, _CH)
    src3b = src[na:].reshape(_NS, cpw1, _CH)
    dst3b = dst[na:].reshape(_NS, cpw1, _CH)
    nh = _NS * cpw * _CH
    ddeg_a = dst[:nh].reshape(_NS, cpw, _CH)
    ddeg_b = dst[nh:].reshape(_NS, cpw, _CH)

    spmm_call = _make_spmm(n, cpw0, cpw1)
    deg_call = _make_deg(n, cpw, cpw)

    degp = deg_call(ddeg_a, ddeg_b)

    h1 = pl.pallas_call(
        _mm1_body,
        grid=grid,
        in_specs=[_row_block(n, d_in), _full_block((d_in, _H))],
        out_specs=_row_block(n, _H),
        out_shape=jax.ShapeDtypeStruct((n, _H), jnp.float32),
    )(x, W1)

    dinvb, g1 = pl.pallas_call(
        _scale_body,
        grid=grid,
        in_specs=[pl.BlockSpec((_NC, _BLK, _H), lambda i: (0, i, 0)),
                  _row_block(n, _H)],
        out_specs=[_row_block(n, _H), _row_block(n, _H)],
        out_shape=[jax.ShapeDtypeStruct((n, _H), jnp.float32),
                   jax.ShapeDtypeStruct((n, _H), jnp.float32)],
    )(degp, h1)

    s1 = spmm_call(g1, src3a, dst3a, src3b, dst3b)

    g2 = pl.pallas_call(
        _mid_body,
        grid=grid,
        in_specs=[pl.BlockSpec((_NC, _BLK, _H), lambda i: (0, i, 0)),
                  _row_block(n, _H), _row_block(n, _H),
                  _full_block((1, _H)), _full_block((_H, _H))],
        out_specs=_row_block(n, _H),
        out_shape=jax.ShapeDtypeStruct((n, _H), jnp.float32),
    )(s1, g1, dinvb, b1.reshape(1, _H), W2)

    s2 = spmm_call(g2, src3a, dst3a, src3b, dst3b)

    c = Wfc.shape[1]
    wpad = jnp.pad(Wfc, ((0, 0), (0, _H - c)))
    bpad = jnp.pad(bfc, (0, _H - c)).reshape(1, _H)
    y = pl.pallas_call(
        _fin_body,
        grid=grid,
        in_specs=[pl.BlockSpec((_NC, _BLK, _H), lambda i: (0, i, 0)),
                  _row_block(n, _H), _row_block(n, _H),
                  _full_block((1, _H)), _full_block((_H, _H)),
                  _full_block((1, _H))],
        out_specs=_row_block(n, _H),
        out_shape=jax.ShapeDtypeStruct((n, _H), jnp.float32),
    )(s2, g2, dinvb, b2.reshape(1, _H), wpad, bpad)

    return y[:, :c]


# speedup vs baseline: 1.1960x; 1.1960x over previous
import functools

import jax
import jax.numpy as jnp
from jax import lax
from jax.experimental import pallas as pl
from jax.experimental.pallas import tpu as pltpu
from jax.experimental.pallas import tpu_sc as plsc

_NC = 2
_NS = 16
_NW = _NC * _NS
_CH = 128
_H = 128

_mesh = plsc.VectorSubcoreMesh(core_axis_name="c", subcore_axis_name="s")


def _zero_rows(buf, nrows, ncols16):
    z = jnp.zeros((16,), jnp.float32)

    def body(i, _):
        r = i // ncols16
        k = i % ncols16
        buf[r, pl.ds(k * 16, 16)] = z
        return 0

    lax.fori_loop(0, nrows * ncols16, body, 0)


_NB = 32


def _make_spmm(n, cpw0, cpw1):
    npad = ((n + 8 * _NS - 1) // (8 * _NS)) * (8 * _NS)
    rpt = npad // _NS

    @functools.partial(
        pl.kernel,
        mesh=_mesh,
        out_type=jax.ShapeDtypeStruct((_NC, npad, _H), jnp.float32),
        scratch_types=[
            pltpu.VMEM((_NB, _CH), jnp.int32),
            pltpu.VMEM((_NB, _CH), jnp.int32),
            pltpu.VMEM((_CH, _H), jnp.float32),
            pltpu.VMEM((_CH, _H), jnp.float32),
            pltpu.VMEM_SHARED((npad, _H), jnp.float32),
            pltpu.SemaphoreType.DMA,
            pltpu.SemaphoreType.DMA,
        ],
    )
    def spmm_kernel(g_hbm, srca_hbm, dsta_hbm, srcb_hbm, dstb_hbm, out_hbm,
                    srcb, dstb, rows_a, rows_b, acc_sh, sem_a, sem_b):
        c = lax.axis_index("c")
        s = lax.axis_index("s")

        _zero_rows(rows_a, _CH, _H // 16)

        nfull = rpt // _CH
        rem = rpt - nfull * _CH
        for k in range(nfull):
            pltpu.sync_copy(rows_a, acc_sh.at[pl.ds(s * rpt + k * _CH, _CH)])
        if rem:
            pltpu.sync_copy(rows_a.at[pl.ds(0, rem)],
                            acc_sh.at[pl.ds(s * rpt + nfull * _CH, rem)])
        plsc.subcore_barrier()

        def gather_a(j):
            pltpu.async_copy(g_hbm.at[srcb.at[j]], rows_a, sem_a)

        def gather_b(j):
            pltpu.async_copy(g_hbm.at[srcb.at[j]], rows_b, sem_b)

        def wait_a(j):
            pltpu.make_async_copy(g_hbm.at[srcb.at[j]], rows_a, sem_a).wait()

        def wait_b(j):
            pltpu.make_async_copy(g_hbm.at[srcb.at[j]], rows_b, sem_b).wait()

        def scat_a(j):
            pltpu.sync_copy(rows_a, acc_sh.at[dstb.at[j]], add=True)

        def scat_b(j):
            pltpu.sync_copy(rows_b, acc_sh.at[dstb.at[j]], add=True)

        def run_edges(src_hbm, dst_hbm, nb):
            def batch_body(b, _):
                pltpu.sync_copy(src_hbm.at[s, pl.ds(b * _NB, _NB)], srcb)
                pltpu.sync_copy(dst_hbm.at[s, pl.ds(b * _NB, _NB)], dstb)

                gather_a(0)

                def pair_body(p, _):
                    gather_b(2 * p + 1)
                    wait_a(2 * p)
                    scat_a(2 * p)
                    gather_a(2 * p + 2)
                    wait_b(2 * p + 1)
                    scat_b(2 * p + 1)
                    return 0

                lax.fori_loop(0, _NB // 2 - 1, pair_body, 0)
                gather_b(_NB - 1)
                wait_a(_NB - 2)
                scat_a(_NB - 2)
                wait_b(_NB - 1)
                scat_b(_NB - 1)
                return 0

            lax.fori_loop(0, nb, batch_body, 0)

        @pl.when(c == 0)
        def _():
            run_edges(srca_hbm, dsta_hbm, cpw0 // _NB)

        @pl.when(c == 1)
        def _():
            run_edges(srcb_hbm, dstb_hbm, cpw1 // _NB)

        plsc.subcore_barrier()

        done = 0
        while done < rpt:
            oc = min(_CH, rpt - done)
            off = s * rpt + done
            pltpu.sync_copy(acc_sh.at[pl.ds(off, oc)], rows_a.at[pl.ds(0, oc)])
            pltpu.sync_copy(rows_a.at[pl.ds(0, oc)], out_hbm.at[c, pl.ds(off, oc)])
            done += oc

    return spmm_kernel


def _make_deg(n, cpw0, cpw1):
    npad = ((n + 8 * _NS - 1) // (8 * _NS)) * (8 * _NS)
    rpt = npad // _NS

    @functools.partial(
        pl.kernel,
        mesh=_mesh,
        out_type=jax.ShapeDtypeStruct((_NC, npad, _H), jnp.float32),
        scratch_types=[
            pltpu.VMEM((_NB, _CH), jnp.int32),
            pltpu.VMEM((_CH, _H), jnp.float32),
            pltpu.VMEM_SHARED((npad, _H), jnp.float32),
        ],
    )
    def deg_kernel(dsta_hbm, dstb_hbm, out_hbm, dstb, ones_v, acc_sh):
        c = lax.axis_index("c")
        s = lax.axis_index("s")

        _zero_rows(ones_v, _CH, _H // 16)

        nfull = rpt // _CH
        rem = rpt - nfull * _CH
        for k in range(nfull):
            pltpu.sync_copy(ones_v, acc_sh.at[pl.ds(s * rpt + k * _CH, _CH)])
        if rem:
            pltpu.sync_copy(ones_v.at[pl.ds(0, rem)],
                            acc_sh.at[pl.ds(s * rpt + nfull * _CH, rem)])

        one = jnp.ones((16,), jnp.float32)

        def fill_ones(i, _):
            r = i // (_H // 16)
            k = i % (_H // 16)
            ones_v[r, pl.ds(k * 16, 16)] = one
            return 0

        lax.fori_loop(0, _CH * (_H // 16), fill_ones, 0)
        plsc.subcore_barrier()

        def run_edges(dst_hbm, nb):
            def batch_body(b, _):
                pltpu.sync_copy(dst_hbm.at[s, pl.ds(b * _NB, _NB)], dstb)

                def body(j, _):
                    pltpu.sync_copy(ones_v, acc_sh.at[dstb.at[j]], add=True)
                    return 0

                lax.fori_loop(0, _NB, body, 0)
                return 0

            lax.fori_loop(0, nb, batch_body, 0)

        @pl.when(c == 0)
        def _():
            run_edges(dsta_hbm, cpw0 // _NB)

        @pl.when(c == 1)
        def _():
            run_edges(dstb_hbm, cpw1 // _NB)

        plsc.subcore_barrier()

        done = 0
        while done < rpt:
            oc = min(_CH, rpt - done)
            off = s * rpt + done
            pltpu.sync_copy(acc_sh.at[pl.ds(off, oc)], ones_v.at[pl.ds(0, oc)])
            pltpu.sync_copy(ones_v.at[pl.ds(0, oc)], out_hbm.at[c, pl.ds(off, oc)])
            done += oc

    return deg_kernel


_BLK = 400


def _mm1_body(x_ref, w_ref, o_ref):
    o_ref[...] = lax.dot_general(
        x_ref[...], w_ref[...], (((1,), (0,)), ((), ())),
        preferred_element_type=jnp.float32, precision=lax.Precision.HIGHEST)


def _scale_body(degp_ref, h_ref, dinv_ref, g_ref):
    deg = degp_ref[0, :, 0:1] + degp_ref[1, :, 0:1] + 1.0
    dinv = lax.rsqrt(deg)
    dinvb = jnp.broadcast_to(dinv, h_ref.shape)
    dinv_ref[...] = dinvb
    g_ref[...] = h_ref[...] * dinvb


def _mid_body(p_ref, g_ref, dinv_ref, b_ref, w_ref, o_ref):
    t = dinv_ref[...] * (p_ref[0] + p_ref[1] + g_ref[...]) + b_ref[...]
    t = jnp.maximum(t, 0.0)
    o_ref[...] = dinv_ref[...] * lax.dot_general(
        t, w_ref[...], (((1,), (0,)), ((), ())),
        preferred_element_type=jnp.float32, precision=lax.Precision.HIGHEST)


def _fin_body(p_ref, g_ref, dinv_ref, b_ref, w_ref, bf_ref, o_ref):
    t = dinv_ref[...] * (p_ref[0] + p_ref[1] + g_ref[...]) + b_ref[...]
    t = jnp.maximum(t, 0.0)
    o_ref[...] = lax.dot_general(
        t, w_ref[...], (((1,), (0,)), ((), ())),
        preferred_element_type=jnp.float32, precision=lax.Precision.HIGHEST) + bf_ref[...]


def _row_block(n, d):
    return pl.BlockSpec((_BLK, d), lambda i: (i, 0))


def _full_block(shape):
    nd = len(shape)
    return pl.BlockSpec(shape, lambda i, _nd=nd: (0,) * nd)


def kernel(x, edge_index, W1, b1, W2, b2, Wfc, bfc):
    n, d_in = x.shape
    e = edge_index.shape[1]
    grid = (n // _BLK,)

    chunks = -(-e // _CH)
    cpw = -(-chunks // _NW)
    cpw = ((cpw + _NB - 1) // _NB) * _NB
    cpw0 = ((2 * cpw * 9 // 10 + _NB - 1) // _NB) * _NB
    cpw1 = 2 * cpw - cpw0
    ep = _NW * cpw * _CH
    src = jnp.concatenate([edge_index[0], jnp.zeros((ep - e,), jnp.int32)])
    dst = jnp.concatenate([edge_index[1], jnp.full((ep - e,), n, jnp.int32)])
    na = _NS * cpw0 * _CH
    src3a = src[:na].reshape(_NS, cpw0, _CH)
    dst3a = dst[:na].reshape(_NS, cpw0, _CH)
    src3b = src[na:].reshape(_NS, cpw1, _CH)
    dst3b = dst[na:].reshape(_NS, cpw1, _CH)
    nh = _NS * cpw * _CH
    ddeg_a = dst[:nh].reshape(_NS, cpw, _CH)
    ddeg_b = dst[nh:].reshape(_NS, cpw, _CH)

    spmm_call = _make_spmm(n, cpw0, cpw1)
    deg_call = _make_deg(n, cpw, cpw)

    degp = deg_call(ddeg_a, ddeg_b)

    h1 = pl.pallas_call(
        _mm1_body,
        grid=grid,
        in_specs=[_row_block(n, d_in), _full_block((d_in, _H))],
        out_specs=_row_block(n, _H),
        out_shape=jax.ShapeDtypeStruct((n, _H), jnp.float32),
    )(x, W1)

    dinvb, g1 = pl.pallas_call(
        _scale_body,
        grid=grid,
        in_specs=[pl.BlockSpec((_NC, _BLK, _H), lambda i: (0, i, 0)),
                  _row_block(n, _H)],
        out_specs=[_row_block(n, _H), _row_block(n, _H)],
        out_shape=[jax.ShapeDtypeStruct((n, _H), jnp.float32),
                   jax.ShapeDtypeStruct((n, _H), jnp.float32)],
    )(degp, h1)

    s1 = spmm_call(g1, src3a, dst3a, src3b, dst3b)

    g2 = pl.pallas_call(
        _mid_body,
        grid=grid,
        in_specs=[pl.BlockSpec((_NC, _BLK, _H), lambda i: (0, i, 0)),
                  _row_block(n, _H), _row_block(n, _H),
                  _full_block((1, _H)), _full_block((_H, _H))],
        out_specs=_row_block(n, _H),
        out_shape=jax.ShapeDtypeStruct((n, _H), jnp.float32),
    )(s1, g1, dinvb, b1.reshape(1, _H), W2)

    s2 = spmm_call(g2, src3a, dst3a, src3b, dst3b)

    c = Wfc.shape[1]
    wpad = jnp.pad(Wfc, ((0, 0), (0, _H - c)))
    bpad = jnp.pad(bfc, (0, _H - c)).reshape(1, _H)
    y = pl.pallas_call(
        _fin_body,
        grid=grid,
        in_specs=[pl.BlockSpec((_NC, _BLK, _H), lambda i: (0, i, 0)),
                  _row_block(n, _H), _row_block(n, _H),
                  _full_block((1, _H)), _full_block((_H, _H)),
                  _full_block((1, _H))],
        out_specs=_row_block(n, _H),
        out_shape=jax.ShapeDtypeStruct((n, _H), jnp.float32),
    )(s2, g2, dinvb, b2.reshape(1, _H), wpad, bpad)

    return y[:, :c]
